# baseline (speedup 1.0000x reference)
import jax
import jax.numpy as jnp
from jax.experimental import pallas as pl
from jax.experimental.pallas import tpu as pltpu

DIM = 1024
HEADS = 16
BUCKET = 64
S = 2048
HD = DIM // HEADS
QBLK = 256
KBLK = 256
NQB = S // QBLK


def _proj_hash_kernel(xq_ref, xk_ref, xv_ref, wq_ref, bq_ref, wk_ref, bk_ref,
                      wv_ref, bv_ref, lsh_ref,
                      Q_ref, K_ref, V_ref, qh_ref, kh_ref):
    lsh = lsh_ref[...]
    q = jnp.dot(xq_ref[...], wq_ref[...],
                preferred_element_type=jnp.float32) + bq_ref[...]
    Q_ref[...] = q
    qh_ref[...] = jnp.argmax(jnp.dot(q, lsh, preferred_element_type=jnp.float32),
                             axis=-1).astype(jnp.int32).reshape(1, QBLK)
    k = jnp.dot(xk_ref[...], wk_ref[...],
                preferred_element_type=jnp.float32) + bk_ref[...]
    K_ref[...] = k
    kh_ref[...] = jnp.argmax(jnp.dot(k, lsh, preferred_element_type=jnp.float32),
                             axis=-1).astype(jnp.int32).reshape(1, QBLK)
    V_ref[...] = jnp.dot(xv_ref[...], wv_ref[...],
                         preferred_element_type=jnp.float32) + bv_ref[...]


def _cumsum_rows(x, n):
    shift = 1
    while shift < n:
        x = x + jnp.concatenate(
            [jnp.zeros((shift, x.shape[1]), x.dtype), x[:n - shift]], axis=0)
        shift *= 2
    return x


def _cumsum_lanes(x, n):
    shift = 1
    while shift < n:
        x = x + jnp.concatenate(
            [jnp.zeros((x.shape[0], shift), x.dtype), x[:, :n - shift]], axis=1)
        shift *= 2
    return x


def _sort_meta_kernel(qh_ref, kh_ref,
                      rankq_ref, rankk_ref, qhs_ref, khs_ref, koff_ref):
    def rank_of(h):
        hb = jax.lax.broadcast_in_dim(h, (S, BUCKET), (0,))
        bid = jax.lax.broadcasted_iota(jnp.int32, (S, BUCKET), 1)
        oh = (hb == bid).astype(jnp.float32)
        incl = _cumsum_rows(oh, S)
        counts = incl[S - 1:S, :]
        incl_cs = _cumsum_lanes(counts, BUCKET)
        offsets = incl_cs - counts
        rank = jnp.sum(oh * (incl - 1.0 + offsets), axis=1)
        return rank, offsets, incl_cs

    rq, _, q_incl_cs = rank_of(qh_ref[0, :])
    rk, k_off, k_incl_cs = rank_of(kh_ref[0, :])
    rankq_ref[...] = rq.astype(jnp.int32).reshape(1, S)
    rankk_ref[...] = rk.astype(jnp.int32).reshape(1, S)

    idx_col = jax.lax.broadcasted_iota(jnp.int32, (S, BUCKET), 0)
    qhs = jnp.sum((idx_col >= jax.lax.broadcast_in_dim(
        q_incl_cs[0].astype(jnp.int32), (S, BUCKET), (1,))).astype(jnp.int32),
        axis=1)
    qhs_ref[...] = qhs.reshape(1, S)
    khs = jnp.sum((idx_col >= jax.lax.broadcast_in_dim(
        k_incl_cs[0].astype(jnp.int32), (S, BUCKET), (1,))).astype(jnp.int32),
        axis=1, keepdims=True)
    khs_ref[...] = khs

    koff_ref[...] = jnp.concatenate(
        [k_off, jnp.full((1, BUCKET), float(S), jnp.float32)],
        axis=1).astype(jnp.int32)


def _gather_kernel(rankq_ref, rankk_ref, Q_ref, K_ref, V_ref,
                   Qs_ref, Ks_ref, Vs_ref):
    i = pl.program_id(0)
    rows = jax.lax.broadcasted_iota(jnp.int32, (QBLK, S), 0) + i * QBLK
    m2q = (rows == jax.lax.broadcast_in_dim(
        rankq_ref[0, :], (QBLK, S), (1,))).astype(jnp.float32)
    m2k = (rows == jax.lax.broadcast_in_dim(
        rankk_ref[0, :], (QBLK, S), (1,))).astype(jnp.float32)
    Qs_ref[...] = jnp.dot(m2q, Q_ref[...], preferred_element_type=jnp.float32)
    Ks_ref[...] = jnp.dot(m2k, K_ref[...], preferred_element_type=jnp.float32)
    Vs_ref[...] = jnp.dot(m2k, V_ref[...], preferred_element_type=jnp.float32)


def _attn_kernel(qhs_v_ref, qhs_s_ref, koff_ref, khs_ref,
                 Qs_ref, Ks_ref, Vs_ref, wo_ref, bo_ref, out_ref):
    qi = pl.program_id(0)
    lo = qhs_s_ref[0, qi * QBLK]
    hi = qhs_s_ref[0, qi * QBLK + QBLK - 1]
    kstart = koff_ref[0, lo]
    kend = koff_ref[0, hi + 1]
    ks0 = (kstart // KBLK) * KBLK
    nch = (kend - ks0 + KBLK - 1) // KBLK

    qh_vec = qhs_v_ref[0, :]
    qhb = jax.lax.broadcast_in_dim(qh_vec, (QBLK, KBLK), (0,))
    q_heads = [Qs_ref[:, h * HD:(h + 1) * HD] for h in range(HEADS)]

    def body(j, carry):
        accs, ms, ls = carry
        ks = ks0 + j * KBLK
        kh_chunk = khs_ref[pl.ds(ks, KBLK), 0]
        mask = qhb == jax.lax.broadcast_in_dim(kh_chunk, (QBLK, KBLK), (1,))
        k_chunk = Ks_ref[pl.ds(ks, KBLK), :]
        v_chunk = Vs_ref[pl.ds(ks, KBLK), :]
        naccs, nms, nls = [], [], []
        for h in range(HEADS):
            sl = slice(h * HD, (h + 1) * HD)
            s = jax.lax.dot_general(
                q_heads[h], k_chunk[:, sl], (((1,), (1,)), ((), ())),
                preferred_element_type=jnp.float32) * 0.125
            s = jnp.where(mask, s, -1e9)
            m_new = jnp.maximum(ms[h], jnp.max(s, axis=-1, keepdims=True))
            corr = jnp.exp(ms[h] - m_new)
            e = jnp.where(mask, jnp.exp(s - m_new), 0.0)
            nls.append(ls[h] * corr + jnp.sum(e, axis=-1, keepdims=True))
            naccs.append(accs[h] * corr +
                         jnp.dot(e, v_chunk[:, sl],
                                 preferred_element_type=jnp.float32))
            nms.append(m_new)
        return naccs, nms, nls

    init = ([jnp.zeros((QBLK, HD), jnp.float32)] * HEADS,
            [jnp.full((QBLK, 1), -1e9, jnp.float32)] * HEADS,
            [jnp.zeros((QBLK, 1), jnp.float32)] * HEADS)
    accs, ms, ls = jax.lax.fori_loop(0, nch, body, init)

    for h in range(HEADS):
        sl = slice(h * HD, (h + 1) * HD)
        has = ls[h] > 0.0
        meanv = jnp.mean(Vs_ref[:, sl], axis=0, keepdims=True)
        row = accs[h] / jnp.where(has, ls[h], 1.0)
        out_ref[:, sl] = jnp.where(
            jax.lax.broadcast_in_dim(has[:, 0], (QBLK, HD), (0,)),
            row, jax.lax.broadcast_in_dim(meanv[0], (QBLK, HD), (1,)))
    out_ref[...] = jnp.dot(out_ref[...], wo_ref[...],
                           preferred_element_type=jnp.float32) + bo_ref[...]


def kernel(query, key, value, Wq, bq, Wk, bk, Wv, bv, Wo, bo, lsh_proj):
    xq, xk, xv = query[0], key[0], value[0]
    bq2, bk2, bv2, bo2 = (b.reshape(1, DIM) for b in (bq, bk, bv, bo))

    full = lambda shape: pl.BlockSpec(shape, lambda i: (0, 0))
    rowblk = pl.BlockSpec((QBLK, DIM), lambda i: (i, 0))
    hashblk = pl.BlockSpec((1, QBLK), lambda i: (0, i))

    Q, K, V, qh, kh = pl.pallas_call(
        _proj_hash_kernel,
        grid=(NQB,),
        in_specs=[rowblk, rowblk, rowblk,
                  full((DIM, DIM)), full((1, DIM)),
                  full((DIM, DIM)), full((1, DIM)),
                  full((DIM, DIM)), full((1, DIM)),
                  full((DIM, BUCKET))],
        out_specs=[rowblk, rowblk, rowblk, hashblk, hashblk],
        out_shape=[jax.ShapeDtypeStruct((S, DIM), jnp.float32),
                   jax.ShapeDtypeStruct((S, DIM), jnp.float32),
                   jax.ShapeDtypeStruct((S, DIM), jnp.float32),
                   jax.ShapeDtypeStruct((1, S), jnp.int32),
                   jax.ShapeDtypeStruct((1, S), jnp.int32)],
    )(xq, xk, xv, Wq, bq2, Wk, bk2, Wv, bv2, lsh_proj)

    rankq, rankk, qhs, khs, koff = pl.pallas_call(
        _sort_meta_kernel,
        grid=(1,),
        in_specs=[full((1, S)), full((1, S))],
        out_specs=[full((1, S)), full((1, S)), full((1, S)),
                   full((S, 1)), full((1, 2 * BUCKET))],
        out_shape=[jax.ShapeDtypeStruct((1, S), jnp.int32),
                   jax.ShapeDtypeStruct((1, S), jnp.int32),
                   jax.ShapeDtypeStruct((1, S), jnp.int32),
                   jax.ShapeDtypeStruct((S, 1), jnp.int32),
                   jax.ShapeDtypeStruct((1, 2 * BUCKET), jnp.int32)],
    )(qh, kh)

    Qs, Ks, Vs = pl.pallas_call(
        _gather_kernel,
        grid=(NQB,),
        in_specs=[full((1, S)), full((1, S)),
                  full((S, DIM)), full((S, DIM)), full((S, DIM))],
        out_specs=[rowblk, rowblk, rowblk],
        out_shape=[jax.ShapeDtypeStruct((S, DIM), jnp.float32)] * 3,
    )(rankq, rankk, Q, K, V)

    out = pl.pallas_call(
        _attn_kernel,
        grid=(NQB,),
        in_specs=[hashblk,
                  pl.BlockSpec(memory_space=pltpu.SMEM),
                  pl.BlockSpec(memory_space=pltpu.SMEM),
                  full((S, 1)),
                  rowblk, full((S, DIM)), full((S, DIM)),
                  full((DIM, DIM)), full((1, DIM))],
        out_specs=rowblk,
        out_shape=jax.ShapeDtypeStruct((S, DIM), jnp.float32),
    )(qhs, qhs, koff, khs, Qs, Ks, Vs, Wo, bo2)

    return out.reshape(1, S, DIM)
